# baseline (speedup 1.0000x reference)
import jax
import jax.numpy as jnp
from jax import lax
from jax.experimental import pallas as pl
from jax.experimental.pallas import tpu as pltpu
from jax.experimental.pallas import tpu_sc as plsc

_B = 16384
_NS = 26
_ND = 13
_NF = _NS + _ND
_V = 100000
_L = 16
_NC = 2
_NSUB = 16
_NW = _NC * _NSUB
_RPT = _B // _NW
_NG = _RPT // _L


def _body(x_hbm, t_hbm, w_hbm, out_hbm, xv, idxv, gv, accv, wv, sem):
    wid = lax.axis_index("s") * _NC + lax.axis_index("c")
    base = wid * _RPT
    pltpu.sync_copy(x_hbm.at[pl.ds(base * _NF, _RPT * _NF)], xv)
    pltpu.sync_copy(w_hbm, wv)
    iota39 = lax.iota(jnp.int32, _L) * _NF

    def build(i, carry):
        pos = iota39 + i * (_L * _NF)
        for f in range(_NS):
            v = plsc.load_gather(xv, [pos + f])
            idxv[pl.ds(f * _RPT + i * _L, _L)] = v.astype(jnp.int32) + f * _V
        return carry

    lax.fori_loop(0, _NG, build, 0)

    pltpu.async_copy(t_hbm.at[idxv], gv, sem).wait()

    def reduce(i, carry):
        pos = iota39 + i * (_L * _NF)
        acc = jnp.zeros((_L,), jnp.float32)
        for f in range(_NS):
            acc = acc + gv[pl.ds(f * _RPT + i * _L, _L)]
        for d in range(_ND):
            acc = acc + plsc.load_gather(xv, [pos + (_NS + d)]) * wv[d]
        accv[pl.ds(i * _L, _L)] = acc
        return carry

    lax.fori_loop(0, _NG, reduce, 0)

    pltpu.sync_copy(accv, out_hbm.at[pl.ds(base, _RPT)])


def kernel(X, emb_tables, dense_weight):
    x_flat = X.reshape(_B * _NF)
    t_flat = emb_tables.reshape(_NS * _V)
    w16 = jnp.broadcast_to(dense_weight.reshape(_ND, 1), (_ND, _L))
    mesh = plsc.VectorSubcoreMesh(core_axis_name="c", subcore_axis_name="s")
    out = pl.kernel(
        _body,
        out_type=jax.ShapeDtypeStruct((_B,), jnp.float32),
        mesh=mesh,
        compiler_params=pltpu.CompilerParams(needs_layout_passes=False),
        scratch_types=[
            pltpu.VMEM((_RPT * _NF,), jnp.float32),
            pltpu.VMEM((_NS * _RPT,), jnp.int32),
            pltpu.VMEM((_NS * _RPT,), jnp.float32),
            pltpu.VMEM((_RPT,), jnp.float32),
            pltpu.VMEM((_ND, _L), jnp.float32),
            pltpu.SemaphoreType.DMA,
        ],
    )(x_flat, t_flat, w16)
    return out.reshape(_B, 1)


# speedup vs baseline: 2.1411x; 2.1411x over previous
import jax
import jax.numpy as jnp
from jax import lax
from jax.experimental import pallas as pl
from jax.experimental.pallas import tpu as pltpu
from jax.experimental.pallas import tpu_sc as plsc

_B = 16384
_NS = 26
_ND = 13
_NF = _NS + _ND
_V = 100000
_L = 16
_NC = 2
_NSUB = 16
_NW = _NC * _NSUB
_RPT = _B // _NW
_NG = _RPT // _L


def _body(*refs):
    xt_hbm = refs[0]
    tabs = refs[1:1 + _NS]
    w_hbm = refs[1 + _NS]
    out_hbm = refs[2 + _NS]
    xv, idxv, gv, accv, wv, sem = refs[3 + _NS:]

    wid = lax.axis_index("s") * _NC + lax.axis_index("c")
    base = wid * _RPT
    pltpu.sync_copy(xt_hbm.at[:, pl.ds(base, _RPT)], xv)
    pltpu.sync_copy(w_hbm, wv)

    def build(i, carry):
        off = i * _L
        for f in range(_NS):
            idxv[f, pl.ds(off, _L)] = xv[f, pl.ds(off, _L)].astype(jnp.int32)
        return carry

    lax.fori_loop(0, _NG, build, 0)

    handles = [
        pltpu.async_copy(tabs[f].at[idxv.at[f]], gv.at[f], sem)
        for f in range(_NS)
    ]
    for h in handles:
        h.wait()

    def reduce(i, carry):
        off = i * _L
        acc = jnp.zeros((_L,), jnp.float32)
        for f in range(_NS):
            acc = acc + gv[f, pl.ds(off, _L)]
        for d in range(_ND):
            acc = acc + xv[_NS + d, pl.ds(off, _L)] * wv[d]
        accv[pl.ds(off, _L)] = acc
        return carry

    lax.fori_loop(0, _NG, reduce, 0)

    pltpu.sync_copy(accv, out_hbm.at[pl.ds(base, _RPT)])


def kernel(X, emb_tables, dense_weight):
    xt = X.T
    tabs = tuple(emb_tables[f, :, 0] for f in range(_NS))
    w16 = jnp.broadcast_to(dense_weight.reshape(_ND, 1), (_ND, _L))
    mesh = plsc.VectorSubcoreMesh(core_axis_name="c", subcore_axis_name="s")
    out = pl.kernel(
        _body,
        out_type=jax.ShapeDtypeStruct((_B,), jnp.float32),
        mesh=mesh,
        compiler_params=pltpu.CompilerParams(
            needs_layout_passes=False, use_tc_tiling_on_sc=False
        ),
        scratch_types=[
            pltpu.VMEM((_NF, _RPT), jnp.float32),
            pltpu.VMEM((_NS, _RPT), jnp.int32),
            pltpu.VMEM((_NS, _RPT), jnp.float32),
            pltpu.VMEM((_RPT,), jnp.float32),
            pltpu.VMEM((_ND, _L), jnp.float32),
            pltpu.SemaphoreType.DMA,
        ],
    )(xt, *tabs, w16)
    return out.reshape(_B, 1)


# speedup vs baseline: 2.2935x; 1.0712x over previous
import jax
import jax.numpy as jnp
from jax import lax
from jax.experimental import pallas as pl
from jax.experimental.pallas import tpu as pltpu
from jax.experimental.pallas import tpu_sc as plsc

_B = 16384
_NS = 26
_ND = 13
_NF = _NS + _ND
_V = 100000
_L = 16
_NC = 2
_NSUB = 16
_NW = _NC * _NSUB
_RPT = _B // _NW
_NG = _RPT // _L
_FC = 13


def _chunk0_body(*refs):
    xt_hbm = refs[0]
    tabs = refs[1:1 + _FC]
    w_hbm = refs[1 + _FC]
    out_hbm = refs[2 + _FC]
    xv, dv, idxv, gv, accv, wv, sem = refs[3 + _FC:]

    wid = lax.axis_index("s") * _NC + lax.axis_index("c")
    base = wid * _RPT
    pltpu.sync_copy(xt_hbm.at[pl.ds(0, _FC), pl.ds(base, _RPT)], xv)
    pltpu.sync_copy(xt_hbm.at[pl.ds(_NS, _ND), pl.ds(base, _RPT)], dv)
    pltpu.sync_copy(w_hbm, wv)

    handles = []
    for f in range(_FC):
        def build(i, carry, f=f):
            off = i * _L
            idxv[f, pl.ds(off, _L)] = xv[f, pl.ds(off, _L)].astype(jnp.int32)
            return carry

        lax.fori_loop(0, _NG, build, 0)
        handles.append(pltpu.async_copy(tabs[f].at[idxv.at[f]], gv.at[f], sem))

    for h in handles:
        h.wait()

    def reduce(i, carry):
        off = i * _L
        acc = jnp.zeros((_L,), jnp.float32)
        for f in range(_FC):
            acc = acc + gv[f, pl.ds(off, _L)]
        for d in range(_ND):
            acc = acc + dv[d, pl.ds(off, _L)] * wv[d]
        accv[pl.ds(off, _L)] = acc
        return carry

    lax.fori_loop(0, _NG, reduce, 0)

    pltpu.sync_copy(accv, out_hbm.at[pl.ds(base, _RPT)])


def _chunk1_body(*refs):
    xt_hbm = refs[0]
    tabs = refs[1:1 + _FC]
    part_hbm = refs[1 + _FC]
    out_hbm = refs[2 + _FC]
    xv, idxv, gv, accv, sem = refs[3 + _FC:]

    wid = lax.axis_index("s") * _NC + lax.axis_index("c")
    base = wid * _RPT
    pltpu.sync_copy(xt_hbm.at[pl.ds(_FC, _FC), pl.ds(base, _RPT)], xv)
    pltpu.sync_copy(part_hbm.at[pl.ds(base, _RPT)], accv)

    handles = []
    for f in range(_FC):
        def build(i, carry, f=f):
            off = i * _L
            idxv[f, pl.ds(off, _L)] = xv[f, pl.ds(off, _L)].astype(jnp.int32)
            return carry

        lax.fori_loop(0, _NG, build, 0)
        handles.append(pltpu.async_copy(tabs[f].at[idxv.at[f]], gv.at[f], sem))

    for h in handles:
        h.wait()

    def reduce(i, carry):
        off = i * _L
        acc = accv[pl.ds(off, _L)]
        for f in range(_FC):
            acc = acc + gv[f, pl.ds(off, _L)]
        accv[pl.ds(off, _L)] = acc
        return carry

    lax.fori_loop(0, _NG, reduce, 0)

    pltpu.sync_copy(accv, out_hbm.at[pl.ds(base, _RPT)])


def kernel(X, emb_tables, dense_weight):
    xt = X.T
    w16 = jnp.broadcast_to(dense_weight.reshape(_ND, 1), (_ND, _L))
    tabs = tuple(emb_tables[f, :, 0] for f in range(_NS))
    mesh = plsc.VectorSubcoreMesh(core_axis_name="c", subcore_axis_name="s")
    params = pltpu.CompilerParams(
        needs_layout_passes=False, use_tc_tiling_on_sc=False
    )
    part = pl.kernel(
        _chunk0_body,
        out_type=jax.ShapeDtypeStruct((_B,), jnp.float32),
        mesh=mesh,
        compiler_params=params,
        scratch_types=[
            pltpu.VMEM((_FC, _RPT), jnp.float32),
            pltpu.VMEM((_ND, _RPT), jnp.float32),
            pltpu.VMEM((_FC, _RPT), jnp.int32),
            pltpu.VMEM((_FC, _RPT), jnp.float32),
            pltpu.VMEM((_RPT,), jnp.float32),
            pltpu.VMEM((_ND, _L), jnp.float32),
            pltpu.SemaphoreType.DMA,
        ],
    )(xt, *tabs[:_FC], w16)
    out = pl.kernel(
        _chunk1_body,
        out_type=jax.ShapeDtypeStruct((_B,), jnp.float32),
        mesh=mesh,
        compiler_params=params,
        scratch_types=[
            pltpu.VMEM((_FC, _RPT), jnp.float32),
            pltpu.VMEM((_FC, _RPT), jnp.int32),
            pltpu.VMEM((_FC, _RPT), jnp.float32),
            pltpu.VMEM((_RPT,), jnp.float32),
            pltpu.SemaphoreType.DMA,
        ],
    )(xt, *tabs[_FC:], part)
    return out.reshape(_B, 1)
